# baseline (speedup 1.0000x reference)
import functools

import jax
import jax.numpy as jnp
from jax import lax
from jax.experimental import pallas as pl
from jax.experimental.pallas import tpu as pltpu
from jax.experimental.pallas import tpu_sc as plsc

NC = 2
NS = 16
L = 16
NW = NC * NS

BATCH = 16384
DIM = 64
BPW = BATCH // NW
CHUNK = 128
NCHUNK = BPW // CHUNK
GPC = CHUNK // L

NROWS = 1000000


def _gmf_body(ut, it, user, item, out,
              uidx_v, iidx_v, uix, iix, ubuf, ibuf, out_v,
              sem0, sem1):
    sems = (sem0, sem1)
    wid = lax.axis_index("s") * NC + lax.axis_index("c")
    base = wid * BPW

    pltpu.sync_copy(user.at[pl.ds(base, BPW)], uidx_v)
    pltpu.sync_copy(item.at[pl.ds(base, BPW)], iidx_v)

    src_u = ut.at[0]
    src_i = it.at[0]

    def fire(j, b):
        def wr(d, carry):
            off = d * NROWS
            for g in range(GPC):
                s = pl.ds(g * L, L)
                uix[b, d, s] = uidx_v[pl.ds(j * CHUNK + g * L, L)] + off
                iix[b, d, s] = iidx_v[pl.ds(j * CHUNK + g * L, L)] + off
            return carry

        lax.fori_loop(0, DIM, wr, 0)
        plsc.subcore_barrier()

        def fd(d, carry):
            pltpu.make_async_copy(
                src_u.at[uix.at[b, d]], ubuf.at[b, d], sems[b]).start()
            pltpu.make_async_copy(
                src_i.at[iix.at[b, d]], ibuf.at[b, d], sems[b]).start()
            return carry

        lax.fori_loop(0, DIM, fd, 0)

    def drain(b):
        def fd(d, carry):
            pltpu.make_async_copy(
                src_u.at[uix.at[b, d]], ubuf.at[b, d], sems[b]).wait()
            pltpu.make_async_copy(
                src_i.at[iix.at[b, d]], ibuf.at[b, d], sems[b]).wait()
            return carry

        lax.fori_loop(0, DIM, fd, 0)

    def compute(j, b):
        for g in range(GPC):
            s = pl.ds(g * L, L)

            def dstep(d, acc):
                return acc + ubuf[b, d, s] * ibuf[b, d, s]

            acc = lax.fori_loop(0, DIM, dstep, jnp.zeros((L,), jnp.float32))
            out_v[pl.ds(j * CHUNK + g * L, L)] = 1.0 / (1.0 + jnp.exp(-acc))

    fire(0, 0)
    for j in range(NCHUNK):
        if j + 1 < NCHUNK:
            fire(j + 1, (j + 1) % 2)
        drain(j % 2)
        compute(j, j % 2)

    pltpu.sync_copy(out_v, out.at[pl.ds(base, BPW)])


_gmf = functools.partial(
    pl.kernel,
    out_type=jax.ShapeDtypeStruct((BATCH,), jnp.float32),
    mesh=plsc.VectorSubcoreMesh(core_axis_name="c", subcore_axis_name="s"),
    scratch_types=[
        pltpu.VMEM((BPW,), jnp.int32),
        pltpu.VMEM((BPW,), jnp.int32),
        pltpu.VMEM((2, DIM, CHUNK), jnp.int32),
        pltpu.VMEM((2, DIM, CHUNK), jnp.int32),
        pltpu.VMEM((2, DIM, CHUNK), jnp.float32),
        pltpu.VMEM((2, DIM, CHUNK), jnp.float32),
        pltpu.VMEM((BPW,), jnp.float32),
        pltpu.SemaphoreType.DMA,
        pltpu.SemaphoreType.DMA,
    ],
    compiler_params=pltpu.CompilerParams(
        needs_layout_passes=False, use_tc_tiling_on_sc=False),
)(_gmf_body)


def kernel(user_table, item_table, user, item):
    return _gmf(user_table.T, item_table.T,
                user.astype(jnp.int32), item.astype(jnp.int32))
